# baseline (speedup 1.0000x reference)
import jax
import jax.numpy as jnp
from jax import lax
from jax.experimental import pallas as pl
from jax.experimental.pallas import tpu as pltpu
from jax.experimental.pallas import tpu_sc as plsc

_N = 10000
_E = 160000
_D = 256
_HALF = _D // 2
_NSUB = 16
_KC = 40
_EPT = _E // _NSUB
_NCH = _EPT // _KC
_R = 4
_WTILES = 10
_ROWS_PT = _N // _WTILES
_ZB = 40

_VOFFS = [0, 16, 24]


def _msgpass_body(x2, ei, ea, msum, cnt, bufs):
    (srcs, dsts, gxs, xrs, ers, ones_v, zb_v, zc_v, acc_sh, cnt_sh,
     sxs, ses, sss, sns, sis, sid) = bufs
    c = lax.axis_index("c")
    s = lax.axis_index("s")

    zeros16 = jnp.zeros((16,), jnp.float32)
    ones16 = jnp.ones((16,), jnp.float32)

    def _zrow(r, carry):
        for j in range(_HALF // 16):
            zb_v[r, pl.ds(j * 16, 16)] = zeros16
        return carry
    lax.fori_loop(0, _ZB, _zrow, 0)
    for o in _VOFFS:
        ones_v[pl.ds(o, 16)] = ones16

    @pl.when(s < _WTILES)
    def _():
        for k in range(_ROWS_PT // _ZB):
            pltpu.sync_copy(zb_v,
                            acc_sh.at[pl.ds(s * _ROWS_PT + k * _ZB, _ZB)])

    @pl.when(jnp.logical_and(c == 0, s == 0))
    def _():
        for j in range(1024 // 16):
            zc_v[pl.ds(j * 16, 16)] = zeros16
        for k in range(_N // 1000):
            pltpu.sync_copy(zc_v.at[pl.ds(0, 1000)],
                            cnt_sh.at[pl.ds(k * 1000, 1000)])

    plsc.subcore_barrier()

    base_e = s * _EPT
    col0 = c * _HALF

    def _build_issue(i, q):
        gb = base_e + i * _KC
        for o in _VOFFS:
            sv = srcs[q][pl.ds(o, 16)]
            gxs[q][pl.ds(o, 16)] = sv * 2 + c
        pltpu.async_copy(x2.at[gxs[q]], xrs[q], sxs[q])
        pltpu.async_copy(ea.at[pl.ds(gb, _KC), pl.ds(col0, _HALF)],
                         ers[q], ses[q])

    def _idx_load_async(i, q):
        gb = base_e + i * _KC
        pltpu.async_copy(ei.at[pl.ds(gb, _KC)], srcs[q], sis[q])
        pltpu.async_copy(ei.at[pl.ds(_E + gb, _KC)], dsts[q], sid[q])

    def _wait_idx(q):
        pltpu.make_async_copy(ei.at[pl.ds(0, _KC)], srcs[q], sis[q]).wait()
        pltpu.make_async_copy(ei.at[pl.ds(0, _KC)], dsts[q], sid[q]).wait()

    def _wait_gathers(q):
        pltpu.make_async_copy(x2.at[gxs[q]], xrs[q], sxs[q]).wait()
        pltpu.make_async_copy(ea.at[pl.ds(0, _KC), pl.ds(0, _HALF)],
                              ers[q], ses[q]).wait()

    def _wait_scatter(q):
        pltpu.make_async_copy(xrs[q], acc_sh.at[dsts[q]], sss[q]).wait()

        @pl.when(c == 0)
        def _():
            pltpu.make_async_copy(ones_v, cnt_sh.at[dsts[q]], sns[q]).wait()

    def _silu_scatter(q):
        def _rows(r, rc):
            for u in range(8):
                for j in range(_HALF // 16):
                    a = xrs[q][r * 8 + u, pl.ds(j * 16, 16)]
                    b = ers[q][r * 8 + u, pl.ds(j * 16, 16)]
                    z = a + b
                    xrs[q][r * 8 + u, pl.ds(j * 16, 16)] = (
                        z / (1.0 + jnp.exp(-z)))
            return rc
        lax.fori_loop(0, _KC // 8, _rows, 0)

        pltpu.async_copy(xrs[q], acc_sh.at[dsts[q]], sss[q], add=True)

        @pl.when(c == 0)
        def _():
            pltpu.async_copy(ones_v, cnt_sh.at[dsts[q]], sns[q], add=True)

    for i0 in range(2):
        gb = base_e + i0 * _KC
        pltpu.sync_copy(ei.at[pl.ds(gb, _KC)], srcs[i0])
        pltpu.sync_copy(ei.at[pl.ds(_E + gb, _KC)], dsts[i0])
        _build_issue(jnp.int32(i0), i0)

    def _step(g, carry):
        for u in range(_R):
            j = _R * g + u
            q = u
            q2 = (u + 2) % _R
            if u < 2:
                @pl.when(g > 0)
                def _():
                    _wait_scatter(q2)
            else:
                _wait_scatter(q2)
            _idx_load_async(j + 2, q2)
            _wait_gathers(q)
            _silu_scatter(q)
            _wait_idx(q2)
            _build_issue(j + 2, q2)
        return carry
    lax.fori_loop(0, _NCH // _R, _step, 0)

    for jt in range(_NCH - 2, _NCH):
        q = jt % _R
        _wait_scatter((jt + 2) % _R)
        _wait_gathers(q)
        _silu_scatter(q)
    _wait_scatter((_NCH - 2) % _R)
    _wait_scatter((_NCH - 1) % _R)

    plsc.subcore_barrier()

    @pl.when(s < _WTILES)
    def _():
        pltpu.sync_copy(acc_sh.at[pl.ds(s * _ROWS_PT, _ROWS_PT)],
                        msum.at[c, pl.ds(s * _ROWS_PT, _ROWS_PT)])

    @pl.when(jnp.logical_and(c == 0, s == 0))
    def _():
        pltpu.sync_copy(cnt_sh, cnt)


def _body_flat(x2, ei, ea, msum, cnt, *scr):
    srcs, dsts, gxs, xrs, ers = (scr[0:4], scr[4:8], scr[8:12],
                                 scr[12:16], scr[16:20])
    ones_v, zb_v, zc_v, acc_sh, cnt_sh = scr[20:25]
    sxs, ses, sss, sns, sis, sid = (scr[25:29], scr[29:33], scr[33:37],
                                    scr[37:41], scr[41:45], scr[45:49])
    _msgpass_body(x2, ei, ea, msum, cnt,
                  (srcs, dsts, gxs, xrs, ers, ones_v, zb_v, zc_v,
                   acc_sh, cnt_sh, sxs, ses, sss, sns, sis, sid))


def _msgpass(x2, ei, ea):
    mesh = plsc.VectorSubcoreMesh(core_axis_name="c", subcore_axis_name="s")
    scratch = (
        [pltpu.VMEM((_KC,), jnp.int32) for _ in range(_R)]
        + [pltpu.VMEM((_KC,), jnp.int32) for _ in range(_R)]
        + [pltpu.VMEM((_KC,), jnp.int32) for _ in range(_R)]
        + [pltpu.VMEM((_KC, _HALF), jnp.float32) for _ in range(_R)]
        + [pltpu.VMEM((_KC, _HALF), jnp.float32) for _ in range(_R)]
        + [
            pltpu.VMEM((_KC,), jnp.float32),
            pltpu.VMEM((_ZB, _HALF), jnp.float32),
            pltpu.VMEM((1024,), jnp.float32),
            pltpu.VMEM_SHARED((_N, _HALF), jnp.float32),
            pltpu.VMEM_SHARED((_N,), jnp.float32),
        ]
        + [pltpu.SemaphoreType.DMA for _ in range(6 * _R)]
    )
    f = pl.kernel(
        _body_flat,
        out_type=[
            jax.ShapeDtypeStruct((2, _N, _HALF), jnp.float32),
            jax.ShapeDtypeStruct((_N,), jnp.float32),
        ],
        mesh=mesh,
        scratch_types=scratch,
    )
    return f(x2, ei, ea)


_BN = 2000


def _mlp_body(x_ref, m_ref, cnt_ref, w1_ref, b1_ref, w2_ref, b2_ref, o_ref):
    xb = x_ref[...]
    m = m_ref[...]
    aggr = jnp.concatenate([m[0], m[1]], axis=-1)
    cntc = jnp.maximum(cnt_ref[...], 1.0)
    z = xb + aggr / cntc
    h = jnp.dot(z.astype(jnp.bfloat16), w1_ref[...].astype(jnp.bfloat16),
                preferred_element_type=jnp.float32) + b1_ref[...]
    h = h / (1.0 + jnp.exp(-h))
    h = jnp.dot(h.astype(jnp.bfloat16), w2_ref[...].astype(jnp.bfloat16),
                preferred_element_type=jnp.float32) + b2_ref[...]
    o_ref[...] = xb + h


def _mlp(x, msum, cnt2, W1, b1, W2, b2):
    grid = (_N // _BN,)
    return pl.pallas_call(
        _mlp_body,
        grid=grid,
        in_specs=[
            pl.BlockSpec((_BN, _D), lambda i: (i, 0)),
            pl.BlockSpec((2, _BN, _HALF), lambda i: (0, i, 0)),
            pl.BlockSpec((_BN, 1), lambda i: (i, 0)),
            pl.BlockSpec((_D, _D), lambda i: (0, 0)),
            pl.BlockSpec((1, _D), lambda i: (0, 0)),
            pl.BlockSpec((_D, _D), lambda i: (0, 0)),
            pl.BlockSpec((1, _D), lambda i: (0, 0)),
        ],
        out_specs=pl.BlockSpec((_BN, _D), lambda i: (i, 0)),
        out_shape=jax.ShapeDtypeStruct((_N, _D), jnp.float32),
    )(x, msum, cnt2, W1, b1, W2, b2)


def kernel(x, edge_index, edge_attr, ln_scale, ln_bias, W1, b1, W2, b2):
    del ln_scale, ln_bias
    x2 = x.reshape(2 * _N, _HALF)
    msum, cnt = _msgpass(x2, edge_index.reshape(2 * _E), edge_attr)
    return _mlp(x, msum, cnt.reshape(_N, 1),
                W1, b1.reshape(1, _D), W2, b2.reshape(1, _D))


# speedup vs baseline: 1.2032x; 1.2032x over previous
import jax
import jax.numpy as jnp
from jax import lax
from jax.experimental import pallas as pl
from jax.experimental.pallas import tpu as pltpu
from jax.experimental.pallas import tpu_sc as plsc

_N = 10000
_E = 160000
_D = 256
_HALF = _D // 2
_NSUB = 16
_KC = 40
_EPT = _E // _NSUB
_NCH = _EPT // _KC
_R = 4
_WTILES = 10
_ROWS_PT = _N // _WTILES
_ZB = 40

_VOFFS = [0, 16, 24]


def _msgpass_body(x2, ei, ea, msum, cnt, bufs):
    (srcs, dsts, gxs, xrs, ers, ones_v, zb_v, zc_v, acc_sh, cnt_sh,
     sxs, ses, sss, sns, sis, sid, sz) = bufs
    c = lax.axis_index("c")
    s = lax.axis_index("s")

    zeros16 = jnp.zeros((16,), jnp.float32)
    ones16 = jnp.ones((16,), jnp.float32)

    def _zrow(r, carry):
        for j in range(_HALF // 16):
            zb_v[r, pl.ds(j * 16, 16)] = zeros16
        return carry
    lax.fori_loop(0, _ZB, _zrow, 0)
    for o in _VOFFS:
        ones_v[pl.ds(o, 16)] = ones16

    base_e = s * _EPT
    col0 = c * _HALF
    for i0 in range(2):
        gb0 = base_e + i0 * _KC
        pltpu.sync_copy(ei.at[pl.ds(gb0, _KC)], srcs[i0])
        pltpu.sync_copy(ei.at[pl.ds(_E + gb0, _KC)], dsts[i0])
        for o in _VOFFS:
            sv = srcs[i0][pl.ds(o, 16)]
            gxs[i0][pl.ds(o, 16)] = sv * 2 + c
        pltpu.async_copy(x2.at[gxs[i0]], xrs[i0], sxs[i0])
        pltpu.async_copy(ea.at[pl.ds(gb0, _KC), pl.ds(col0, _HALF)],
                         ers[i0], ses[i0])

    @pl.when(s < _WTILES)
    def _():
        for k in range(_ROWS_PT // _ZB):
            pltpu.async_copy(
                zb_v, acc_sh.at[pl.ds(s * _ROWS_PT + k * _ZB, _ZB)], sz)
        for k in range(_ROWS_PT // _ZB):
            pltpu.make_async_copy(
                zb_v, acc_sh.at[pl.ds(s * _ROWS_PT + k * _ZB, _ZB)],
                sz).wait()

    @pl.when(jnp.logical_and(c == 0, s == 0))
    def _():
        for j in range(1024 // 16):
            zc_v[pl.ds(j * 16, 16)] = zeros16
        for k in range(_N // 1000):
            pltpu.sync_copy(zc_v.at[pl.ds(0, 1000)],
                            cnt_sh.at[pl.ds(k * 1000, 1000)])

    plsc.subcore_barrier()

    def _build_issue(i, q):
        gb = base_e + i * _KC
        for o in _VOFFS:
            sv = srcs[q][pl.ds(o, 16)]
            gxs[q][pl.ds(o, 16)] = sv * 2 + c
        pltpu.async_copy(x2.at[gxs[q]], xrs[q], sxs[q])
        pltpu.async_copy(ea.at[pl.ds(gb, _KC), pl.ds(col0, _HALF)],
                         ers[q], ses[q])

    def _idx_load_async(i, q):
        gb = base_e + i * _KC
        pltpu.async_copy(ei.at[pl.ds(gb, _KC)], srcs[q], sis[q])
        pltpu.async_copy(ei.at[pl.ds(_E + gb, _KC)], dsts[q], sid[q])

    def _wait_idx(q):
        pltpu.make_async_copy(ei.at[pl.ds(0, _KC)], srcs[q], sis[q]).wait()
        pltpu.make_async_copy(ei.at[pl.ds(0, _KC)], dsts[q], sid[q]).wait()

    def _wait_gathers(q):
        pltpu.make_async_copy(x2.at[gxs[q]], xrs[q], sxs[q]).wait()
        pltpu.make_async_copy(ea.at[pl.ds(0, _KC), pl.ds(0, _HALF)],
                              ers[q], ses[q]).wait()

    def _wait_scatter(q):
        pltpu.make_async_copy(xrs[q], acc_sh.at[dsts[q]], sss[q]).wait()

        @pl.when(c == 0)
        def _():
            pltpu.make_async_copy(ones_v, cnt_sh.at[dsts[q]], sns[q]).wait()

    def _silu_scatter(q):
        def _rows(r, rc):
            for u in range(4):
                for j in range(_HALF // 16):
                    a = xrs[q][r * 4 + u, pl.ds(j * 16, 16)]
                    b = ers[q][r * 4 + u, pl.ds(j * 16, 16)]
                    z = a + b
                    xrs[q][r * 4 + u, pl.ds(j * 16, 16)] = (
                        z / (1.0 + jnp.exp(-z)))
            return rc
        lax.fori_loop(0, _KC // 4, _rows, 0)

        pltpu.async_copy(xrs[q], acc_sh.at[dsts[q]], sss[q], add=True)

        @pl.when(c == 0)
        def _():
            pltpu.async_copy(ones_v, cnt_sh.at[dsts[q]], sns[q], add=True)

    def _step(g, carry):
        for u in range(_R):
            j = _R * g + u
            q = u
            q2 = (u + 2) % _R
            if u < 2:
                @pl.when(g > 0)
                def _():
                    _wait_scatter(q2)
            else:
                _wait_scatter(q2)
            _idx_load_async(j + 2, q2)
            _wait_gathers(q)
            _silu_scatter(q)
            _wait_idx(q2)
            _build_issue(j + 2, q2)
        return carry
    lax.fori_loop(0, _NCH // _R, _step, 0)

    for jt in range(_NCH - 2, _NCH):
        q = jt % _R
        _wait_scatter((jt + 2) % _R)
        _wait_gathers(q)
        _silu_scatter(q)
    _wait_scatter((_NCH - 2) % _R)
    _wait_scatter((_NCH - 1) % _R)

    plsc.subcore_barrier()

    @pl.when(s < _WTILES)
    def _():
        pltpu.sync_copy(acc_sh.at[pl.ds(s * _ROWS_PT, _ROWS_PT)],
                        msum.at[c, pl.ds(s * _ROWS_PT, _ROWS_PT)])

    @pl.when(jnp.logical_and(c == 0, s == 0))
    def _():
        pltpu.sync_copy(cnt_sh, cnt)


def _body_flat(x2, ei, ea, msum, cnt, *scr):
    srcs, dsts, gxs, xrs, ers = (scr[0:4], scr[4:8], scr[8:12],
                                 scr[12:16], scr[16:20])
    ones_v, zb_v, zc_v, acc_sh, cnt_sh = scr[20:25]
    sxs, ses, sss, sns, sis, sid = (scr[25:29], scr[29:33], scr[33:37],
                                    scr[37:41], scr[41:45], scr[45:49])
    sz = scr[49]
    _msgpass_body(x2, ei, ea, msum, cnt,
                  (srcs, dsts, gxs, xrs, ers, ones_v, zb_v, zc_v,
                   acc_sh, cnt_sh, sxs, ses, sss, sns, sis, sid, sz))


def _msgpass(x2, ei, ea):
    mesh = plsc.VectorSubcoreMesh(core_axis_name="c", subcore_axis_name="s")
    scratch = (
        [pltpu.VMEM((_KC,), jnp.int32) for _ in range(_R)]
        + [pltpu.VMEM((_KC,), jnp.int32) for _ in range(_R)]
        + [pltpu.VMEM((_KC,), jnp.int32) for _ in range(_R)]
        + [pltpu.VMEM((_KC, _HALF), jnp.float32) for _ in range(_R)]
        + [pltpu.VMEM((_KC, _HALF), jnp.float32) for _ in range(_R)]
        + [
            pltpu.VMEM((_KC,), jnp.float32),
            pltpu.VMEM((_ZB, _HALF), jnp.float32),
            pltpu.VMEM((1024,), jnp.float32),
            pltpu.VMEM_SHARED((_N, _HALF), jnp.float32),
            pltpu.VMEM_SHARED((_N,), jnp.float32),
        ]
        + [pltpu.SemaphoreType.DMA for _ in range(6 * _R + 1)]
    )
    f = pl.kernel(
        _body_flat,
        out_type=[
            jax.ShapeDtypeStruct((2, _N, _HALF), jnp.float32),
            jax.ShapeDtypeStruct((_N,), jnp.float32),
        ],
        mesh=mesh,
        scratch_types=scratch,
    )
    return f(x2, ei, ea)


_BN = 2000


def _mlp_body(x_ref, m_ref, cnt_ref, w1_ref, b1_ref, w2_ref, b2_ref, o_ref):
    xb = x_ref[...]
    m = m_ref[...]
    aggr = jnp.concatenate([m[0], m[1]], axis=-1)
    cntc = jnp.maximum(cnt_ref[...], 1.0)
    z = xb + aggr / cntc
    h = jnp.dot(z.astype(jnp.bfloat16), w1_ref[...].astype(jnp.bfloat16),
                preferred_element_type=jnp.float32) + b1_ref[...]
    h = h / (1.0 + jnp.exp(-h))
    h = jnp.dot(h.astype(jnp.bfloat16), w2_ref[...].astype(jnp.bfloat16),
                preferred_element_type=jnp.float32) + b2_ref[...]
    o_ref[...] = xb + h


def _mlp(x, msum, cnt2, W1, b1, W2, b2):
    grid = (_N // _BN,)
    return pl.pallas_call(
        _mlp_body,
        grid=grid,
        in_specs=[
            pl.BlockSpec((_BN, _D), lambda i: (i, 0)),
            pl.BlockSpec((2, _BN, _HALF), lambda i: (0, i, 0)),
            pl.BlockSpec((_BN, 1), lambda i: (i, 0)),
            pl.BlockSpec((_D, _D), lambda i: (0, 0)),
            pl.BlockSpec((1, _D), lambda i: (0, 0)),
            pl.BlockSpec((_D, _D), lambda i: (0, 0)),
            pl.BlockSpec((1, _D), lambda i: (0, 0)),
        ],
        out_specs=pl.BlockSpec((_BN, _D), lambda i: (i, 0)),
        out_shape=jax.ShapeDtypeStruct((_N, _D), jnp.float32),
    )(x, msum, cnt2, W1, b1, W2, b2)


def kernel(x, edge_index, edge_attr, ln_scale, ln_bias, W1, b1, W2, b2):
    del ln_scale, ln_bias
    x2 = x.reshape(2 * _N, _HALF)
    msum, cnt = _msgpass(x2, edge_index.reshape(2 * _E), edge_attr)
    return _mlp(x, msum, cnt.reshape(_N, 1),
                W1, b1.reshape(1, _D), W2, b2.reshape(1, _D))
